# baseline (speedup 1.0000x reference)
import functools

import jax
import jax.numpy as jnp
from jax import lax
from jax.experimental import pallas as pl
from jax.experimental.pallas import tpu as pltpu
from jax.experimental.pallas import tpu_sc as plsc

N = 50000
E = 800000
DIN = 16
DE = 4
DH = 32
DO = 16

NC = 2
NS = 16
NW = NC * NS
EPW = E // NW
BLK = 200
NFULL = EPW // BLK
TAIL = EPW - NFULL * BLK
TPAD = TAIL + 8
BPAD = BLK + 8
CHUNK = 3128
NPAD = CHUNK * NS
LASTC = N - (NS - 1) * CHUNK

_MESH = plsc.VectorSubcoreMesh(
    core_axis_name="c", subcore_axis_name="s", num_cores=NC, num_subcores=NS)
_SC_PARAMS = pltpu.CompilerParams(use_tc_tiling_on_sc=False)


def _make_sc_layer(dh):
  nv = dh // 16
  wlen = DE * dh + dh

  out_type = [
      jax.ShapeDtypeStruct((NC, N, dh), jnp.float32),
      jax.ShapeDtypeStruct((NC * NPAD,), jnp.float32),
  ]

  scratch = [
      pltpu.VMEM_SHARED((NPAD, dh), jnp.float32),
      pltpu.VMEM_SHARED((NPAD,), jnp.float32),
  ]
  scratch += [
      pltpu.VMEM((2, BLK), jnp.int32),
      pltpu.VMEM((BLK,), jnp.int32),
      pltpu.VMEM((DE, BPAD), jnp.float32),
      pltpu.VMEM((BLK, dh), jnp.float32),
      pltpu.VMEM((BLK, dh), jnp.float32),
      pltpu.VMEM((BLK, dh), jnp.float32),
      pltpu.VMEM((BPAD,), jnp.float32),
  ]
  if TAIL:
    scratch += [
        pltpu.VMEM((TAIL,), jnp.int32),
        pltpu.VMEM((TAIL,), jnp.int32),
        [pltpu.VMEM((TPAD,), jnp.float32) for _ in range(DE)],
        pltpu.VMEM((TAIL, dh), jnp.float32),
        pltpu.VMEM((TAIL, dh), jnp.float32),
        pltpu.VMEM((TAIL, dh), jnp.float32),
        pltpu.VMEM((TPAD,), jnp.float32),
    ]
  scratch += [
      pltpu.VMEM((wlen,), jnp.float32),
      pltpu.SemaphoreType.DMA,
      pltpu.SemaphoreType.DMA,
  ]

  def body(*refs):
    n_in = 7
    ins, outs, scr = refs[:n_in], refs[n_in:n_in + 2], refs[n_in + 2:]
    (xl_hbm, xr_hbm, ei_hbm, eat_hbm, wcat_hbm, zx_hbm, z1_hbm) = ins
    x_out, d_out = outs
    x_acc, d_acc = scr[0], scr[1]
    bufs = scr[2:]
    (ei_b, dsc_b, ea_b, rl_b, rr_b, sc_b, al_b, wbuf, sem, sem2) = bufs

    c = lax.axis_index("c")
    s = lax.axis_index("s")
    wid = s * NC + c
    iota = lax.iota(jnp.int32, 16)
    bfly = [iota ^ sh for sh in (8, 4, 2, 1)]

    pltpu.sync_copy(wcat_hbm, wbuf)
    wv = [[wbuf[pl.ds(k * dh + j * 16, 16)] for j in range(nv)]
          for k in range(DE)]
    attv = [wbuf[pl.ds(DE * dh + j * 16, 16)] for j in range(nv)]

    zs = s * CHUNK
    pltpu.sync_copy(zx_hbm.at[pl.ds(zs, CHUNK)], x_acc.at[pl.ds(zs, CHUNK)])
    pltpu.sync_copy(z1_hbm.at[pl.ds(zs, CHUNK)], d_acc.at[pl.ds(zs, CHUNK)])
    plsc.subcore_barrier()

    def group_compute(gbase, ns_, eakr, rlr, rrr, scr_ref, alr):
      avs = [eakr[k, pl.ds(gbase, 16)] for k in range(DE)]
      alv = jnp.zeros((16,), jnp.float32)
      for si in range(ns_):
        e = gbase + si
        vl = [rlr[e, pl.ds(16 * j, 16)] for j in range(nv)]
        vr = [rrr[e, pl.ds(16 * j, 16)] for j in range(nv)]
        t = None
        for j in range(nv):
          v = vl[j] + vr[j]
          for k in range(DE):
            v = v + jnp.full((16,), avs[k][si]) * wv[k][j]
          v = jnp.maximum(v, 0.2 * v)
          t = v * attv[j] if t is None else t + v * attv[j]
        for b in bfly:
          t = t + jnp.take(t, b)
        al = jnp.exp(t)
        for j in range(nv):
          scr_ref[e, pl.ds(16 * j, 16)] = al * vl[j]
        alv = jnp.where(iota == si, al, alv)
      alr[pl.ds(gbase, 16)] = alv

    def process(base):
      loads = [
          pltpu.async_copy(ei_hbm.at[:, pl.ds(base, BLK)], ei_b, sem),
          pltpu.async_copy(eat_hbm.at[:, pl.ds(base, BLK)],
                           ea_b.at[:, pl.ds(0, BLK)], sem),
      ]
      for d in loads:
        d.wait()
      gathers = [
          pltpu.async_copy(xl_hbm.at[ei_b.at[0]], rl_b, sem2),
          pltpu.async_copy(xr_hbm.at[ei_b.at[1]], rr_b, sem2),
      ]
      for d in gathers:
        d.wait()

      for off in list(range(0, BLK - 16, 16)) + [BLK - 16]:
        dsc_b[pl.ds(off, 16)] = ei_b[1, pl.ds(off, 16)]

      ngf = BLK // 16

      @pl.loop(0, ngf)
      def _(g):
        group_compute(g * 16, 16, ea_b, rl_b, rr_b, sc_b, al_b)

      rem = BLK - ngf * 16
      if rem:
        group_compute(ngf * 16, rem, ea_b, rl_b, rr_b, sc_b, al_b)

      scats = [
          pltpu.async_copy(sc_b, x_acc.at[dsc_b], sem, add=True),
          pltpu.async_copy(al_b.at[pl.ds(0, BLK)], d_acc.at[dsc_b],
                           sem, add=True),
      ]
      for d in scats:
        d.wait()

    ebase = wid * EPW

    @pl.loop(0, NFULL)
    def _(i):
      process(ebase + i * BLK)

    plsc.subcore_barrier()

    fs = s * CHUNK

    def flush(flen):
      pltpu.sync_copy(x_acc.at[pl.ds(fs, flen)],
                      x_out.at[c, pl.ds(fs, flen)])
      pltpu.sync_copy(d_acc.at[pl.ds(fs, flen)],
                      d_out.at[pl.ds(c * NPAD + fs, flen)])

    @pl.when(s < NS - 1)
    def _():
      flush(CHUNK)

    @pl.when(s == NS - 1)
    def _():
      flush(LASTC)

  return pl.kernel(body, out_type=out_type, mesh=_MESH,
                   scratch_types=scratch, compiler_params=_SC_PARAMS)


def _make_sc_degattr():
  out_type = [jax.ShapeDtypeStruct((NC * NPAD,), jnp.float32)
              for _ in range(1 + DE)]
  scratch = (
      [pltpu.VMEM_SHARED((NPAD,), jnp.float32) for _ in range(1 + DE)] + [
          pltpu.VMEM((2, BLK), jnp.int32),
          pltpu.VMEM((BLK,), jnp.int32),
          pltpu.VMEM((DE, BLK), jnp.float32),
          [pltpu.VMEM((BLK,), jnp.float32) for _ in range(DE)],
          pltpu.VMEM((BPAD,), jnp.float32),
          pltpu.SemaphoreType.DMA,
      ])

  def body(ei_hbm, eat_hbm, z1_hbm,
           deg_out, as0_out, as1_out, as2_out, as3_out,
           deg_acc, a0_acc, a1_acc, a2_acc, a3_acc,
           ei_b, dst_b, ea_b, eak_b, ones_b, sem):
    accs = (deg_acc, a0_acc, a1_acc, a2_acc, a3_acc)
    outs = (deg_out, as0_out, as1_out, as2_out, as3_out)

    c = lax.axis_index("c")
    s = lax.axis_index("s")
    wid = s * NC + c

    for i in range(BPAD // 16):
      ones_b[pl.ds(i * 16, 16)] = jnp.full((16,), 1.0, jnp.float32)

    zs = s * CHUNK
    for acc in accs:
      pltpu.sync_copy(z1_hbm.at[pl.ds(zs, CHUNK)], acc.at[pl.ds(zs, CHUNK)])
    plsc.subcore_barrier()

    def process(base):
      loads = [
          pltpu.async_copy(ei_hbm.at[:, pl.ds(base, BLK)], ei_b, sem),
          pltpu.async_copy(eat_hbm.at[:, pl.ds(base, BLK)], ea_b, sem),
      ]
      for d in loads:
        d.wait()
      for off in list(range(0, BLK - 16, 16)) + [BLK - 16]:
        dst_b[pl.ds(off, 16)] = ei_b[1, pl.ds(off, 16)]
      for k in range(DE):
        for off in list(range(0, BLK - 16, 16)) + [BLK - 16]:
          eak_b[k][pl.ds(off, 16)] = ea_b[k, pl.ds(off, 16)]
      scats = [pltpu.async_copy(ones_b.at[pl.ds(0, BLK)], deg_acc.at[dst_b],
                                sem, add=True)]
      scats += [
          pltpu.async_copy(eak_b[k], accs[1 + k].at[dst_b], sem, add=True)
          for k in range(DE)
      ]
      for d in scats:
        d.wait()

    ebase = wid * EPW

    @pl.loop(0, NFULL)
    def _(i):
      process(ebase + i * BLK)

    plsc.subcore_barrier()
    fs = s * CHUNK

    def flush(flen):
      for acc, o in zip(accs, outs):
        pltpu.sync_copy(acc.at[pl.ds(fs, flen)],
                        o.at[pl.ds(c * NPAD + fs, flen)])

    @pl.when(s < NS - 1)
    def _():
      flush(CHUNK)

    @pl.when(s == NS - 1)
    def _():
      flush(LASTC)

  return pl.kernel(body, out_type=out_type, mesh=_MESH,
                   scratch_types=scratch, compiler_params=_SC_PARAMS)


_sc_layer1 = _make_sc_layer(DH)
_sc_layer2 = _make_sc_layer(DO)
_sc_degattr = _make_sc_degattr()

RB = 1000
GRID = N // RB


def _tc_pre(x, wl, bl, wr, br):
  def body(x_ref, wl_ref, bl_ref, wr_ref, br_ref, xl_ref, xr_ref):
    xb = x_ref[...]
    xl_ref[...] = jnp.dot(xb, wl_ref[...],
                          preferred_element_type=jnp.float32) + bl_ref[...]
    xr_ref[...] = jnp.dot(xb, wr_ref[...],
                          preferred_element_type=jnp.float32) + br_ref[...]

  full = lambda shape: pl.BlockSpec(shape, lambda i: (0,) * len(shape))
  return pl.pallas_call(
      body,
      grid=(GRID,),
      in_specs=[
          pl.BlockSpec((RB, DIN), lambda i: (i, 0)),
          full((DIN, DH)), full((1, DH)), full((DIN, DH)), full((1, DH)),
      ],
      out_specs=[pl.BlockSpec((RB, DH), lambda i: (i, 0))] * 2,
      out_shape=[jax.ShapeDtypeStruct((N, DH), jnp.float32)] * 2,
  )(x, wl, bl.reshape(1, DH), wr, br.reshape(1, DH))


def _tc_mid(xl1, xr1, x1p, d1p, degp, asp, we1, att1, bias1,
            wl2, bl2, wr2, br2, we2, att2):
  def body(xl1_r, xr1_r, x1p_r, d1p_r, degp_r, as0_r, as1_r, as2_r, as3_r,
           we1_r, att1_r, bias1_r, wl2_r, bl2_r, wr2_r, br2_r, we2_r,
           att2_r, xl2_o, xr2_o, als2_o):
    dot = functools.partial(jnp.dot, preferred_element_type=jnp.float32)
    xl = xl1_r[...]
    xr = xr1_r[...]
    x1p = x1p_r[...]
    d1p = d1p_r[...]
    degp = degp_r[...]
    deg = jnp.maximum(degp[0] + degp[1], 1.0)
    ask = [r[...] for r in (as0_r, as1_r, as2_r, as3_r)]
    lak = [(a[0] + a[1]) / deg for a in ask]
    we1v = we1_r[...]
    we2v = we2_r[...]
    lw1 = sum(lak[k] * we1v[k:k + 1, :] for k in range(DE))
    v = xl + xr + lw1
    v = jnp.maximum(v, 0.2 * v)
    als1 = jnp.exp(dot(v, att1_r[...]))
    d = d1p[0] + d1p[1] + als1
    xacc = x1p[0] + x1p[1] + als1 * xl
    h = jnp.maximum(xacc / d + bias1_r[...], 0.0)
    xl2 = dot(h, wl2_r[...]) + bl2_r[...]
    xr2 = dot(h, wr2_r[...]) + br2_r[...]
    lw2 = sum(lak[k] * we2v[k:k + 1, :] for k in range(DE))
    v2 = xl2 + xr2 + lw2
    v2 = jnp.maximum(v2, 0.2 * v2)
    als2_o[...] = jnp.exp(dot(v2, att2_r[...]))
    xl2_o[...] = xl2
    xr2_o[...] = xr2

  full = lambda shape: pl.BlockSpec(shape, lambda i: (0,) * len(shape))
  np1 = pl.BlockSpec((NC, RB, 1), lambda i: (0, i, 0))
  return pl.pallas_call(
      body,
      grid=(GRID,),
      in_specs=[
          pl.BlockSpec((RB, DH), lambda i: (i, 0)),
          pl.BlockSpec((RB, DH), lambda i: (i, 0)),
          pl.BlockSpec((NC, RB, DH), lambda i: (0, i, 0)),
          np1, np1, np1, np1, np1, np1,
          full((DE, DH)), full((DH, 1)), full((1, DH)),
          full((DH, DO)), full((1, DO)), full((DH, DO)), full((1, DO)),
          full((DE, DO)), full((DO, 1)),
      ],
      out_specs=[
          pl.BlockSpec((RB, DO), lambda i: (i, 0)),
          pl.BlockSpec((RB, DO), lambda i: (i, 0)),
          pl.BlockSpec((RB, 1), lambda i: (i, 0)),
      ],
      out_shape=[
          jax.ShapeDtypeStruct((N, DO), jnp.float32),
          jax.ShapeDtypeStruct((N, DO), jnp.float32),
          jax.ShapeDtypeStruct((N, 1), jnp.float32),
      ],
  )(xl1, xr1, x1p, d1p, degp, *asp, we1, att1, bias1,
    wl2, bl2, wr2, br2, we2, att2)


def _tc_post(x2p, d2p, als2, xl2, bias2):
  def body(x2p_r, d2p_r, als2_r, xl2_r, bias2_r, out_r):
    x2p = x2p_r[...]
    d2p = d2p_r[...]
    als2 = als2_r[...]
    xl2 = xl2_r[...]
    d = d2p[0] + d2p[1] + als2
    xacc = x2p[0] + x2p[1] + als2 * xl2
    out_r[...] = xacc / d + bias2_r[...]

  full = lambda shape: pl.BlockSpec(shape, lambda i: (0,) * len(shape))
  return pl.pallas_call(
      body,
      grid=(GRID,),
      in_specs=[
          pl.BlockSpec((NC, RB, DO), lambda i: (0, i, 0)),
          pl.BlockSpec((NC, RB, 1), lambda i: (0, i, 0)),
          pl.BlockSpec((RB, 1), lambda i: (i, 0)),
          pl.BlockSpec((RB, DO), lambda i: (i, 0)),
          full((1, DO)),
      ],
      out_specs=pl.BlockSpec((RB, DO), lambda i: (i, 0)),
      out_shape=jax.ShapeDtypeStruct((N, DO), jnp.float32),
  )(x2p, d2p, als2, xl2, bias2)


def _unflat(a):
  return a.reshape(NC, NPAD)[:, :N].reshape(NC, N, 1)


def kernel(x, edge_index, edge_attr, Wl1, bl1, Wr1, br1, We1, att1, bias1,
           Wl2, bl2, Wr2, br2, We2, att2, bias2):
  xl1, xr1 = _tc_pre(x, Wl1, bl1, Wr1, br1)

  wcat1 = jnp.concatenate([We1.reshape(-1), att1])
  wcat2 = jnp.concatenate([We2.reshape(-1), att2])
  zx1 = jnp.zeros((NPAD, DH), jnp.float32)
  zx2 = jnp.zeros((NPAD, DO), jnp.float32)
  z1 = jnp.zeros((NPAD,), jnp.float32)

  eaT = edge_attr.T
  degp, *asp = _sc_degattr(edge_index, eaT, z1)
  x1p, d1p = _sc_layer1(xl1, xr1, edge_index, eaT, wcat1, zx1, z1)

  xl2, xr2, als2 = _tc_mid(
      xl1, xr1, x1p, _unflat(d1p), _unflat(degp), [_unflat(a) for a in asp],
      We1, att1.reshape(DH, 1), bias1.reshape(1, DH),
      Wl2, bl2.reshape(1, DO), Wr2, br2.reshape(1, DO),
      We2, att2.reshape(DO, 1))

  x2p, d2p = _sc_layer2(xl2, xr2, edge_index, eaT, wcat2, zx2, z1)

  out = _tc_post(x2p, _unflat(d2p), als2, xl2, bias2.reshape(1, DO))
  return out.reshape(-1)


# speedup vs baseline: 1.0549x; 1.0549x over previous
import functools

import jax
import jax.numpy as jnp
from jax import lax
from jax.experimental import pallas as pl
from jax.experimental.pallas import tpu as pltpu
from jax.experimental.pallas import tpu_sc as plsc

N = 50000
E = 800000
DIN = 16
DE = 4
DH = 32
DO = 16

NC = 2
NS = 16
NW = NC * NS
EPW = E // NW
BLK = 200
NFULL = EPW // BLK
TAIL = EPW - NFULL * BLK
TPAD = TAIL + 8
BPAD = BLK + 8
CHUNK = 3128
NPAD = CHUNK * NS
LASTC = N - (NS - 1) * CHUNK

_MESH = plsc.VectorSubcoreMesh(
    core_axis_name="c", subcore_axis_name="s", num_cores=NC, num_subcores=NS)
_SC_PARAMS = pltpu.CompilerParams(use_tc_tiling_on_sc=False)


def _make_sc_layer(dh):
  nv = dh // 16
  wlen = DE * dh + dh

  out_type = [
      jax.ShapeDtypeStruct((NC, N, dh), jnp.float32),
      jax.ShapeDtypeStruct((NC * NPAD,), jnp.float32),
  ]

  scratch = [
      pltpu.VMEM_SHARED((NPAD, dh), jnp.float32),
      pltpu.VMEM_SHARED((NPAD,), jnp.float32),
  ]
  scratch += [
      pltpu.VMEM((2, BLK), jnp.int32),
      pltpu.VMEM((BLK,), jnp.int32),
      pltpu.VMEM((DE, BPAD), jnp.float32),
      pltpu.VMEM((BLK, dh), jnp.float32),
      pltpu.VMEM((BLK, dh), jnp.float32),
      pltpu.VMEM((BLK, dh), jnp.float32),
      pltpu.VMEM((BPAD,), jnp.float32),
  ]
  if TAIL:
    scratch += [
        pltpu.VMEM((TAIL,), jnp.int32),
        pltpu.VMEM((TAIL,), jnp.int32),
        [pltpu.VMEM((TPAD,), jnp.float32) for _ in range(DE)],
        pltpu.VMEM((TAIL, dh), jnp.float32),
        pltpu.VMEM((TAIL, dh), jnp.float32),
        pltpu.VMEM((TAIL, dh), jnp.float32),
        pltpu.VMEM((TPAD,), jnp.float32),
    ]
  scratch += [
      pltpu.VMEM((wlen,), jnp.float32),
      pltpu.SemaphoreType.DMA,
      pltpu.SemaphoreType.DMA,
  ]

  def body(*refs):
    n_in = 7
    ins, outs, scr = refs[:n_in], refs[n_in:n_in + 2], refs[n_in + 2:]
    (xl_hbm, xr_hbm, ei_hbm, eat_hbm, wcat_hbm, zx_hbm, z1_hbm) = ins
    x_out, d_out = outs
    x_acc, d_acc = scr[0], scr[1]
    bufs = scr[2:]
    (ei_b, dsc_b, ea_b, rl_b, rr_b, sc_b, al_b, wbuf, sem, sem2) = bufs

    c = lax.axis_index("c")
    s = lax.axis_index("s")
    wid = s * NC + c
    iota = lax.iota(jnp.int32, 16)
    bfly = [iota ^ sh for sh in (8, 4, 2, 1)]

    pltpu.sync_copy(wcat_hbm, wbuf)
    wv = [[wbuf[pl.ds(k * dh + j * 16, 16)] for j in range(nv)]
          for k in range(DE)]
    attv = [wbuf[pl.ds(DE * dh + j * 16, 16)] for j in range(nv)]

    zs = s * CHUNK
    pltpu.sync_copy(zx_hbm.at[pl.ds(zs, CHUNK)], x_acc.at[pl.ds(zs, CHUNK)])
    pltpu.sync_copy(z1_hbm.at[pl.ds(zs, CHUNK)], d_acc.at[pl.ds(zs, CHUNK)])
    plsc.subcore_barrier()

    def group_compute(gbase, ns_, eakr, rlr, rrr, scr_ref, alr):
      avs = [eakr[k, pl.ds(gbase, 16)] for k in range(DE)]
      alv = jnp.zeros((16,), jnp.float32)
      for si in range(ns_):
        e = gbase + si
        vl = [rlr[e, pl.ds(16 * j, 16)] for j in range(nv)]
        vr = [rrr[e, pl.ds(16 * j, 16)] for j in range(nv)]
        t = None
        for j in range(nv):
          v = vl[j] + vr[j]
          for k in range(DE):
            v = v + jnp.full((16,), avs[k][si]) * wv[k][j]
          v = jnp.maximum(v, 0.2 * v)
          t = v * attv[j] if t is None else t + v * attv[j]
        for b in bfly:
          t = t + jnp.take(t, b)
        al = jnp.exp(t)
        for j in range(nv):
          scr_ref[e, pl.ds(16 * j, 16)] = al * vl[j]
        alv = jnp.where(iota == si, al, alv)
      alr[pl.ds(gbase, 16)] = alv

    def process(base):
      loads = [
          pltpu.async_copy(ei_hbm.at[:, pl.ds(base, BLK)], ei_b, sem),
          pltpu.async_copy(eat_hbm.at[:, pl.ds(base, BLK)],
                           ea_b.at[:, pl.ds(0, BLK)], sem),
      ]
      for d in loads:
        d.wait()
      gathers = [
          pltpu.async_copy(xl_hbm.at[ei_b.at[0]], rl_b, sem2),
          pltpu.async_copy(xr_hbm.at[ei_b.at[1]], rr_b, sem2),
      ]
      for d in gathers:
        d.wait()

      for off in list(range(0, BLK - 16, 16)) + [BLK - 16]:
        dsc_b[pl.ds(off, 16)] = ei_b[1, pl.ds(off, 16)]

      ngf = BLK // 16

      @pl.loop(0, ngf)
      def _(g):
        group_compute(g * 16, 16, ea_b, rl_b, rr_b, sc_b, al_b)

      rem = BLK - ngf * 16
      if rem:
        group_compute(ngf * 16, rem, ea_b, rl_b, rr_b, sc_b, al_b)

      scats = [
          pltpu.async_copy(sc_b, x_acc.at[dsc_b], sem, add=True),
          pltpu.async_copy(al_b.at[pl.ds(0, BLK)], d_acc.at[dsc_b],
                           sem, add=True),
      ]
      for d in scats:
        d.wait()

    ebase = wid * EPW

    @pl.loop(0, NFULL)
    def _(i):
      process(ebase + i * BLK)

    plsc.subcore_barrier()

    fs = s * CHUNK

    def flush(flen):
      pltpu.sync_copy(x_acc.at[pl.ds(fs, flen)],
                      x_out.at[c, pl.ds(fs, flen)])
      pltpu.sync_copy(d_acc.at[pl.ds(fs, flen)],
                      d_out.at[pl.ds(c * NPAD + fs, flen)])

    @pl.when(s < NS - 1)
    def _():
      flush(CHUNK)

    @pl.when(s == NS - 1)
    def _():
      flush(LASTC)

  return pl.kernel(body, out_type=out_type, mesh=_MESH,
                   scratch_types=scratch, compiler_params=_SC_PARAMS)


def _make_sc_degattr():
  out_type = [jax.ShapeDtypeStruct((NC * NPAD,), jnp.float32)
              for _ in range(1 + DE)]
  scratch = (
      [pltpu.VMEM_SHARED((NPAD,), jnp.float32) for _ in range(1 + DE)] + [
          pltpu.VMEM((BLK,), jnp.int32),
          [pltpu.VMEM((BLK,), jnp.float32) for _ in range(DE)],
          pltpu.VMEM((BPAD,), jnp.float32),
          pltpu.SemaphoreType.DMA,
      ])

  def body(dst_hbm, ea0_hbm, ea1_hbm, ea2_hbm, ea3_hbm, z1_hbm,
           deg_out, as0_out, as1_out, as2_out, as3_out,
           deg_acc, a0_acc, a1_acc, a2_acc, a3_acc,
           dst_b, eak_b, ones_b, sem):
    eak_hbm = (ea0_hbm, ea1_hbm, ea2_hbm, ea3_hbm)
    accs = (deg_acc, a0_acc, a1_acc, a2_acc, a3_acc)
    outs = (deg_out, as0_out, as1_out, as2_out, as3_out)

    c = lax.axis_index("c")
    s = lax.axis_index("s")
    wid = s * NC + c

    for i in range(BPAD // 16):
      ones_b[pl.ds(i * 16, 16)] = jnp.full((16,), 1.0, jnp.float32)

    zs = s * CHUNK
    for acc in accs:
      pltpu.sync_copy(z1_hbm.at[pl.ds(zs, CHUNK)], acc.at[pl.ds(zs, CHUNK)])
    plsc.subcore_barrier()

    def process(nedges, dstr, eakr, base):
      sub = lambda ref: (ref if ref.shape[0] == nedges
                         else ref.at[pl.ds(0, nedges)])
      loads = [pltpu.async_copy(dst_hbm.at[pl.ds(base, nedges)], dstr, sem)]
      loads += [
          pltpu.async_copy(eak_hbm[k].at[pl.ds(base, nedges)], eakr[k], sem)
          for k in range(DE)
      ]
      for d in loads:
        d.wait()
      scats = [pltpu.async_copy(sub(ones_b), deg_acc.at[dstr], sem,
                                add=True)]
      scats += [
          pltpu.async_copy(eakr[k], accs[1 + k].at[dstr], sem, add=True)
          for k in range(DE)
      ]
      for d in scats:
        d.wait()

    ebase = wid * EPW

    @pl.loop(0, NFULL)
    def _(i):
      process(BLK, dst_b, eak_b, ebase + i * BLK)

    plsc.subcore_barrier()
    fs = s * CHUNK

    def flush(flen):
      for acc, o in zip(accs, outs):
        pltpu.sync_copy(acc.at[pl.ds(fs, flen)],
                        o.at[pl.ds(c * NPAD + fs, flen)])

    @pl.when(s < NS - 1)
    def _():
      flush(CHUNK)

    @pl.when(s == NS - 1)
    def _():
      flush(LASTC)

  return pl.kernel(body, out_type=out_type, mesh=_MESH,
                   scratch_types=scratch, compiler_params=_SC_PARAMS)


_sc_layer1 = _make_sc_layer(DH)
_sc_layer2 = _make_sc_layer(DO)
_sc_degattr = _make_sc_degattr()

RB = 1000
GRID = N // RB


def _tc_pre(x, wl, bl, wr, br):
  def body(x_ref, wl_ref, bl_ref, wr_ref, br_ref, xl_ref, xr_ref):
    xb = x_ref[...]
    xl_ref[...] = jnp.dot(xb, wl_ref[...],
                          preferred_element_type=jnp.float32) + bl_ref[...]
    xr_ref[...] = jnp.dot(xb, wr_ref[...],
                          preferred_element_type=jnp.float32) + br_ref[...]

  full = lambda shape: pl.BlockSpec(shape, lambda i: (0,) * len(shape))
  return pl.pallas_call(
      body,
      grid=(GRID,),
      in_specs=[
          pl.BlockSpec((RB, DIN), lambda i: (i, 0)),
          full((DIN, DH)), full((1, DH)), full((DIN, DH)), full((1, DH)),
      ],
      out_specs=[pl.BlockSpec((RB, DH), lambda i: (i, 0))] * 2,
      out_shape=[jax.ShapeDtypeStruct((N, DH), jnp.float32)] * 2,
  )(x, wl, bl.reshape(1, DH), wr, br.reshape(1, DH))


def _tc_mid(xl1, xr1, x1p, d1p, degp, asp, we1, att1, bias1,
            wl2, bl2, wr2, br2, we2, att2):
  def body(xl1_r, xr1_r, x1p_r, d1p_r, degp_r, as0_r, as1_r, as2_r, as3_r,
           we1_r, att1_r, bias1_r, wl2_r, bl2_r, wr2_r, br2_r, we2_r,
           att2_r, xl2_o, xr2_o, als2_o):
    dot = functools.partial(jnp.dot, preferred_element_type=jnp.float32)
    xl = xl1_r[...]
    xr = xr1_r[...]
    x1p = x1p_r[...]
    d1p = d1p_r[...]
    degp = degp_r[...]
    deg = jnp.maximum(degp[0] + degp[1], 1.0)
    ask = [r[...] for r in (as0_r, as1_r, as2_r, as3_r)]
    lak = [(a[0] + a[1]) / deg for a in ask]
    we1v = we1_r[...]
    we2v = we2_r[...]
    lw1 = sum(lak[k] * we1v[k:k + 1, :] for k in range(DE))
    v = xl + xr + lw1
    v = jnp.maximum(v, 0.2 * v)
    als1 = jnp.exp(dot(v, att1_r[...]))
    d = d1p[0] + d1p[1] + als1
    xacc = x1p[0] + x1p[1] + als1 * xl
    h = jnp.maximum(xacc / d + bias1_r[...], 0.0)
    xl2 = dot(h, wl2_r[...]) + bl2_r[...]
    xr2 = dot(h, wr2_r[...]) + br2_r[...]
    lw2 = sum(lak[k] * we2v[k:k + 1, :] for k in range(DE))
    v2 = xl2 + xr2 + lw2
    v2 = jnp.maximum(v2, 0.2 * v2)
    als2_o[...] = jnp.exp(dot(v2, att2_r[...]))
    xl2_o[...] = xl2
    xr2_o[...] = xr2

  full = lambda shape: pl.BlockSpec(shape, lambda i: (0,) * len(shape))
  np1 = pl.BlockSpec((NC, RB, 1), lambda i: (0, i, 0))
  return pl.pallas_call(
      body,
      grid=(GRID,),
      in_specs=[
          pl.BlockSpec((RB, DH), lambda i: (i, 0)),
          pl.BlockSpec((RB, DH), lambda i: (i, 0)),
          pl.BlockSpec((NC, RB, DH), lambda i: (0, i, 0)),
          np1, np1, np1, np1, np1, np1,
          full((DE, DH)), full((DH, 1)), full((1, DH)),
          full((DH, DO)), full((1, DO)), full((DH, DO)), full((1, DO)),
          full((DE, DO)), full((DO, 1)),
      ],
      out_specs=[
          pl.BlockSpec((RB, DO), lambda i: (i, 0)),
          pl.BlockSpec((RB, DO), lambda i: (i, 0)),
          pl.BlockSpec((RB, 1), lambda i: (i, 0)),
      ],
      out_shape=[
          jax.ShapeDtypeStruct((N, DO), jnp.float32),
          jax.ShapeDtypeStruct((N, DO), jnp.float32),
          jax.ShapeDtypeStruct((N, 1), jnp.float32),
      ],
  )(xl1, xr1, x1p, d1p, degp, *asp, we1, att1, bias1,
    wl2, bl2, wr2, br2, we2, att2)


def _tc_post(x2p, d2p, als2, xl2, bias2):
  def body(x2p_r, d2p_r, als2_r, xl2_r, bias2_r, out_r):
    x2p = x2p_r[...]
    d2p = d2p_r[...]
    als2 = als2_r[...]
    xl2 = xl2_r[...]
    d = d2p[0] + d2p[1] + als2
    xacc = x2p[0] + x2p[1] + als2 * xl2
    out_r[...] = xacc / d + bias2_r[...]

  full = lambda shape: pl.BlockSpec(shape, lambda i: (0,) * len(shape))
  return pl.pallas_call(
      body,
      grid=(GRID,),
      in_specs=[
          pl.BlockSpec((NC, RB, DO), lambda i: (0, i, 0)),
          pl.BlockSpec((NC, RB, 1), lambda i: (0, i, 0)),
          pl.BlockSpec((RB, 1), lambda i: (i, 0)),
          pl.BlockSpec((RB, DO), lambda i: (i, 0)),
          full((1, DO)),
      ],
      out_specs=pl.BlockSpec((RB, DO), lambda i: (i, 0)),
      out_shape=jax.ShapeDtypeStruct((N, DO), jnp.float32),
  )(x2p, d2p, als2, xl2, bias2)


def _unflat(a):
  return a.reshape(NC, NPAD)[:, :N].reshape(NC, N, 1)


def kernel(x, edge_index, edge_attr, Wl1, bl1, Wr1, br1, We1, att1, bias1,
           Wl2, bl2, Wr2, br2, We2, att2, bias2):
  src = edge_index[0]
  dst = edge_index[1]

  xl1, xr1 = _tc_pre(x, Wl1, bl1, Wr1, br1)

  wcat1 = jnp.concatenate([We1.reshape(-1), att1])
  wcat2 = jnp.concatenate([We2.reshape(-1), att2])
  zx1 = jnp.zeros((NPAD, DH), jnp.float32)
  zx2 = jnp.zeros((NPAD, DO), jnp.float32)
  z1 = jnp.zeros((NPAD,), jnp.float32)

  ea_cols = [edge_attr[:, k] for k in range(DE)]
  eaT = edge_attr.T
  degp, *asp = _sc_degattr(dst, *ea_cols, z1)
  x1p, d1p = _sc_layer1(xl1, xr1, edge_index, eaT, wcat1, zx1, z1)

  xl2, xr2, als2 = _tc_mid(
      xl1, xr1, x1p, _unflat(d1p), _unflat(degp), [_unflat(a) for a in asp],
      We1, att1.reshape(DH, 1), bias1.reshape(1, DH),
      Wl2, bl2.reshape(1, DO), Wr2, br2.reshape(1, DO),
      We2, att2.reshape(DO, 1))

  x2p, d2p = _sc_layer2(xl2, xr2, edge_index, eaT, wcat2, zx2, z1)

  out = _tc_post(x2p, _unflat(d2p), als2, xl2, bias2.reshape(1, DO))
  return out.reshape(-1)
